# Initial kernel scaffold; baseline (speedup 1.0000x reference)
import jax
import jax.numpy as jnp
from jax.experimental import pallas as pl


def kernel(indices, table):
    raise NotImplementedError("write your pallas kernel here")


# speedup vs baseline: 1.5578x; 1.5578x over previous
import functools

import jax
import jax.numpy as jnp
from jax import lax
from jax.experimental import pallas as pl
from jax.experimental.pallas import tpu as pltpu
from jax.experimental.pallas import tpu_sc as plsc

_NC = 2
_NS = 16
_NW = _NC * _NS

_CHUNK = 128
_GROUP = 8


@functools.lru_cache(maxsize=None)
def _build(bf: int, d: int):
    b_per_w = bf // _NW
    n_chunks = b_per_w // _CHUNK
    n_groups = n_chunks // _GROUP
    rows_per_group = _GROUP * _CHUNK

    mesh = plsc.VectorSubcoreMesh(core_axis_name="c", subcore_axis_name="s")

    @functools.partial(
        pl.kernel,
        out_type=jax.ShapeDtypeStruct((bf, d), jnp.float32),
        mesh=mesh,
        scratch_types=[
            pltpu.VMEM((n_chunks, _CHUNK), jnp.int32),
            pltpu.VMEM((rows_per_group, d), jnp.float32),
            pltpu.SemaphoreType.DMA,
        ],
        compiler_params=pltpu.CompilerParams(use_tc_tiling_on_sc=False),
    )
    def gather_kernel(idx_hbm, table_hbm, out_hbm, idx_v, rows_v, gsem):
        wid = lax.axis_index("s") * _NC + lax.axis_index("c")
        base = wid * b_per_w
        pltpu.sync_copy(idx_hbm.at[wid], idx_v)

        @pl.loop(0, n_groups)
        def _(g):
            handles = [
                pltpu.async_copy(
                    table_hbm.at[idx_v.at[g * _GROUP + j]],
                    rows_v.at[pl.ds(j * _CHUNK, _CHUNK)],
                    gsem,
                )
                for j in range(_GROUP)
            ]
            for h in handles:
                h.wait()
            pltpu.sync_copy(
                rows_v, out_hbm.at[pl.ds(base + g * rows_per_group, rows_per_group)]
            )

    return gather_kernel


@jax.jit
def kernel(indices, table):
    b, f = indices.shape
    _, d = table.shape
    bf = b * f
    idx = indices.astype(jnp.int32).reshape(_NW, bf // (_NW * _CHUNK), _CHUNK)
    out = _build(bf, d)(idx, table)
    return out.reshape(b, f, d)


# speedup vs baseline: 1.5772x; 1.0124x over previous
import functools

import jax
import jax.numpy as jnp
from jax import lax
from jax.experimental import pallas as pl
from jax.experimental.pallas import tpu as pltpu
from jax.experimental.pallas import tpu_sc as plsc

_NC = 2
_NS = 16
_NW = _NC * _NS

_CHUNK = 128
_GROUP = 8


@functools.lru_cache(maxsize=None)
def _build(bf: int, d: int):
    b_per_w = bf // _NW
    n_chunks = b_per_w // _CHUNK
    n_groups = n_chunks // _GROUP
    rows_per_group = _GROUP * _CHUNK

    mesh = plsc.VectorSubcoreMesh(core_axis_name="c", subcore_axis_name="s")

    @functools.partial(
        pl.kernel,
        out_type=jax.ShapeDtypeStruct((bf, d), jnp.float32),
        mesh=mesh,
        scratch_types=[
            pltpu.VMEM((n_chunks, _CHUNK), jnp.int32),
            pltpu.VMEM((2, rows_per_group, d), jnp.float32),
            pltpu.SemaphoreType.DMA,
            pltpu.SemaphoreType.DMA,
        ],
        compiler_params=pltpu.CompilerParams(use_tc_tiling_on_sc=False),
    )
    def gather_kernel(idx_hbm, table_hbm, out_hbm, idx_v, rows_v, gsem, wsem):
        wid = lax.axis_index("s") * _NC + lax.axis_index("c")
        base = wid * b_per_w
        pltpu.sync_copy(idx_hbm.at[wid], idx_v)

        def fire(g, buf):
            for j in range(_GROUP):
                pltpu.async_copy(
                    table_hbm.at[idx_v.at[g * _GROUP + j]],
                    buf.at[pl.ds(j * _CHUNK, _CHUNK)],
                    gsem,
                )

        def drain(sem, b):
            pltpu.make_async_copy(
                out_hbm.at[pl.ds(base, rows_per_group)], rows_v.at[b], sem
            ).wait()

        fire(0, rows_v.at[0])

        @pl.loop(0, n_groups)
        def _(g):
            cur = g % 2
            @pl.when(g > 0)
            def _():
                drain(wsem, 1 - cur)

            @pl.when(g + 1 < n_groups)
            def _():
                fire(g + 1, rows_v.at[1 - cur])

            drain(gsem, cur)
            pltpu.async_copy(
                rows_v.at[cur],
                out_hbm.at[pl.ds(base + g * rows_per_group, rows_per_group)],
                wsem,
            )

        drain(wsem, (n_groups - 1) % 2)

    return gather_kernel


@jax.jit
def kernel(indices, table):
    b, f = indices.shape
    _, d = table.shape
    bf = b * f
    idx = indices.astype(jnp.int32).reshape(_NW, bf // (_NW * _CHUNK), _CHUNK)
    out = _build(bf, d)(idx, table)
    return out.reshape(b, f, d)


# speedup vs baseline: 1.5775x; 1.0002x over previous
import functools

import jax
import jax.numpy as jnp
from jax import lax
from jax.experimental import pallas as pl
from jax.experimental.pallas import tpu as pltpu
from jax.experimental.pallas import tpu_sc as plsc

_NC = 2
_NS = 16
_NW = _NC * _NS

_CHUNK = 128
_GROUP = 8


@functools.lru_cache(maxsize=None)
def _build(bf: int, d: int):
    b_per_w = bf // _NW
    n_chunks = b_per_w // _CHUNK
    n_groups = n_chunks // _GROUP
    rows_per_group = _GROUP * _CHUNK

    mesh = plsc.VectorSubcoreMesh(core_axis_name="c", subcore_axis_name="s")

    @functools.partial(
        pl.kernel,
        out_type=jax.ShapeDtypeStruct((bf, d), jnp.float32),
        mesh=mesh,
        scratch_types=[
            pltpu.VMEM((n_chunks, _CHUNK), jnp.int32),
            pltpu.VMEM((2, rows_per_group, d), jnp.float32),
        ]
        + [pltpu.SemaphoreType.DMA] * _GROUP
        + [pltpu.SemaphoreType.DMA],
        compiler_params=pltpu.CompilerParams(use_tc_tiling_on_sc=False),
    )
    def gather_kernel(idx_hbm, table_hbm, out_hbm, idx_v, rows_v, *sems):
        gsems = sems[:_GROUP]
        wsem = sems[_GROUP]
        wid = lax.axis_index("s") * _NC + lax.axis_index("c")
        base = wid * b_per_w
        pltpu.sync_copy(idx_hbm.at[wid], idx_v)

        def fire(g, buf):
            for j in range(_GROUP):
                pltpu.async_copy(
                    table_hbm.at[idx_v.at[g * _GROUP + j]],
                    buf.at[pl.ds(j * _CHUNK, _CHUNK)],
                    gsems[j],
                )

        def drain_gathers(b):
            for j in range(_GROUP):
                pltpu.make_async_copy(
                    out_hbm.at[pl.ds(base, _CHUNK)],
                    rows_v.at[b, pl.ds(j * _CHUNK, _CHUNK)],
                    gsems[j],
                ).wait()

        def drain(sem, b):
            pltpu.make_async_copy(
                out_hbm.at[pl.ds(base, rows_per_group)], rows_v.at[b], sem
            ).wait()

        fire(0, rows_v.at[0])

        @pl.loop(0, n_groups)
        def _(g):
            cur = g % 2
            @pl.when(g > 0)
            def _():
                drain(wsem, 1 - cur)

            @pl.when(g + 1 < n_groups)
            def _():
                fire(g + 1, rows_v.at[1 - cur])

            drain_gathers(cur)
            pltpu.async_copy(
                rows_v.at[cur],
                out_hbm.at[pl.ds(base + g * rows_per_group, rows_per_group)],
                wsem,
            )

        drain(wsem, (n_groups - 1) % 2)

    return gather_kernel


@jax.jit
def kernel(indices, table):
    b, f = indices.shape
    _, d = table.shape
    bf = b * f
    idx = indices.astype(jnp.int32).reshape(_NW, bf // (_NW * _CHUNK), _CHUNK)
    out = _build(bf, d)(idx, table)
    return out.reshape(b, f, d)
